# baseline (speedup 1.0000x reference)
import functools

import jax
import jax.numpy as jnp
from jax import lax
from jax.experimental import pallas as pl
from jax.experimental.pallas import tpu as pltpu
from jax.experimental.pallas import tpu_sc as plsc

NC = 2
NS = 16
L = 16
NW = NC * NS
CHUNK = 128
PHROWS = 256


def kernel(user_table, item_table, user_idx, item_idx):
    B = user_idx.shape[0]
    D = user_table.shape[1]
    bpw = B // NW
    nchunk = bpw // CHUNK
    nphase = bpw // PHROWS
    cpp = PHROWS // CHUNK

    ui32 = user_idx.astype(jnp.int32)
    ii32 = item_idx.astype(jnp.int32)
    ub2 = (ui32 >> 1).reshape(NW * nchunk, CHUNK)
    ib2 = (ii32 >> 1).reshape(NW * nchunk, CHUNK)
    uh = ui32 & 1
    ih = ii32 & 1
    ut2 = user_table.reshape(-1, 2 * D)
    it2 = item_table.reshape(-1, 2 * D)

    mesh = plsc.VectorSubcoreMesh(core_axis_name="c", subcore_axis_name="s")

    @functools.partial(
        pl.kernel,
        mesh=mesh,
        compiler_params=pltpu.CompilerParams(needs_layout_passes=False),
        out_type=jax.ShapeDtypeStruct((B,), jnp.float32),
        scratch_types=[
            pltpu.VMEM((nchunk, CHUNK), jnp.int32),
            pltpu.VMEM((nchunk, CHUNK), jnp.int32),
            pltpu.VMEM((bpw,), jnp.int32),
            pltpu.VMEM((bpw,), jnp.int32),
            pltpu.VMEM((PHROWS, 2 * D), jnp.float32),
            pltpu.VMEM((PHROWS, 2 * D), jnp.float32),
            pltpu.VMEM((bpw,), jnp.float32),
            pltpu.SemaphoreType.DMA,
            pltpu.SemaphoreType.DMA,
        ],
    )
    def _k(ut_hbm, it_hbm, ub_hbm, ib_hbm, uh_hbm, ih_hbm, out_hbm,
           ub_v, ib_v, uh_v, ih_v, ur_v, ir_v, o_v, sem_u, sem_i):
        wid = lax.axis_index("s") * NC + lax.axis_index("c")
        pltpu.sync_copy(ub_hbm.at[pl.ds(wid * nchunk, nchunk)], ub_v)
        pltpu.sync_copy(ib_hbm.at[pl.ds(wid * nchunk, nchunk)], ib_v)
        pltpu.sync_copy(uh_hbm.at[pl.ds(wid * bpw, bpw)], uh_v)
        pltpu.sync_copy(ih_hbm.at[pl.ds(wid * bpw, bpw)], ih_v)

        for h in range(nphase):
            copies = []
            for j in range(cpp):
                copies.append(pltpu.async_copy(
                    ut_hbm.at[ub_v.at[h * cpp + j]],
                    ur_v.at[pl.ds(j * CHUNK, CHUNK)], sem_u))
                copies.append(pltpu.async_copy(
                    it_hbm.at[ib_v.at[h * cpp + j]],
                    ir_v.at[pl.ds(j * CHUNK, CHUNK)], sem_i))
            for c in copies:
                c.wait()

            def group(g, carry):
                rows = g * L + lax.broadcasted_iota(jnp.int32, (L,), 0)
                boff = h * PHROWS + g * L
                ucol = uh_v[pl.ds(boff, L)] * D
                icol = ih_v[pl.ds(boff, L)] * D
                acc = jnp.zeros((L,), jnp.float32)
                for d in range(D):
                    u = plsc.load_gather(ur_v, [rows, ucol + d])
                    v = plsc.load_gather(ir_v, [rows, icol + d])
                    acc = acc + u * v
                o_v[pl.ds(boff, L)] = acc
                return carry

            lax.fori_loop(0, PHROWS // L, group, 0)

        pltpu.sync_copy(o_v, out_hbm.at[pl.ds(wid * bpw, bpw)])

    return _k(ut2, it2, ub2, ib2, uh, ih)


# speedup vs baseline: 1.6208x; 1.6208x over previous
import functools

import jax
import jax.numpy as jnp
from jax import lax
from jax.experimental import pallas as pl
from jax.experimental.pallas import tpu as pltpu
from jax.experimental.pallas import tpu_sc as plsc

NC = 2
NS = 16
L = 16
NW = NC * NS
PHROWS = 256


def _lane(vec, k):
    return lax.squeeze(lax.slice(vec, (k,), (k + 1,)), (0,))


def kernel(user_table, item_table, user_idx, item_idx):
    B = user_idx.shape[0]
    D = user_table.shape[1]
    bpw = B // NW
    nphase = bpw // PHROWS

    ui32 = user_idx.astype(jnp.int32)
    ii32 = item_idx.astype(jnp.int32)

    mesh = plsc.VectorSubcoreMesh(core_axis_name="c", subcore_axis_name="s")

    @functools.partial(
        pl.kernel,
        mesh=mesh,
        compiler_params=pltpu.CompilerParams(needs_layout_passes=False),
        out_type=jax.ShapeDtypeStruct((B,), jnp.float32),
        scratch_types=[
            pltpu.VMEM((bpw,), jnp.int32),
            pltpu.VMEM((bpw,), jnp.int32),
            pltpu.VMEM((PHROWS, D), jnp.float32),
            pltpu.VMEM((PHROWS, D), jnp.float32),
            pltpu.VMEM((bpw,), jnp.float32),
            pltpu.SemaphoreType.DMA,
            pltpu.SemaphoreType.DMA,
        ],
    )
    def _k(ut_hbm, it_hbm, ui_hbm, ii_hbm, out_hbm,
           ui_v, ii_v, ur_v, ir_v, o_v, sem_u, sem_i):
        wid = lax.axis_index("s") * NC + lax.axis_index("c")
        pltpu.sync_copy(ui_hbm.at[pl.ds(wid * bpw, bpw)], ui_v)
        pltpu.sync_copy(ii_hbm.at[pl.ds(wid * bpw, bpw)], ii_v)

        lanes = lax.broadcasted_iota(jnp.int32, (L,), 0)

        for h in range(nphase):
            for g in range(PHROWS // L):
                uvec = ui_v[pl.ds(h * PHROWS + g * L, L)]
                ivec = ii_v[pl.ds(h * PHROWS + g * L, L)]
                for k in range(L):
                    r = g * L + k
                    pltpu.async_copy(ut_hbm.at[_lane(uvec, k)], ur_v.at[r], sem_u)
                    pltpu.async_copy(it_hbm.at[_lane(ivec, k)], ir_v.at[r], sem_i)

            def drain(r, carry):
                pltpu.make_async_copy(ut_hbm.at[0], ur_v.at[r], sem_u).wait()
                pltpu.make_async_copy(it_hbm.at[0], ir_v.at[r], sem_i).wait()
                return carry

            lax.fori_loop(0, PHROWS, drain, 0)

            def group(g, carry):
                out = jnp.zeros((L,), jnp.float32)
                for k in range(L):
                    r = g * L + k
                    acc = jnp.zeros((L,), jnp.float32)
                    for c in range(D // L):
                        u = ur_v[r, pl.ds(c * L, L)]
                        v = ir_v[r, pl.ds(c * L, L)]
                        acc = acc + u * v
                    out = jnp.where(lanes == k, jnp.sum(acc), out)
                o_v[pl.ds(h * PHROWS + g * L, L)] = out
                return carry

            lax.fori_loop(0, PHROWS // L, group, 0)

        pltpu.sync_copy(o_v, out_hbm.at[pl.ds(wid * bpw, bpw)])

    return _k(user_table, item_table, ui32, ii32)


# speedup vs baseline: 2.3849x; 1.4714x over previous
import functools

import jax
import jax.numpy as jnp
from jax import lax
from jax.experimental import pallas as pl
from jax.experimental.pallas import tpu as pltpu
from jax.experimental.pallas import tpu_sc as plsc

NC = 2
NS = 16
L = 16
NW = NC * NS
CHUNK = 128
PHROWS = 256
TCC = 4096


def _repack_kernel(u, i, uo, io):
    uo[:, :64] = u[:, :TCC].T
    uo[:, 64:] = u[:, TCC:].T
    io[:, :64] = i[:, :TCC].T
    io[:, 64:] = i[:, TCC:].T


def _repack(ut_t, it_t):
    D, N = ut_t.shape
    grid = (N // 2 + TCC - 1) // TCC
    H = grid * TCC
    return pl.pallas_call(
        _repack_kernel,
        grid=(grid,),
        in_specs=[
            pl.BlockSpec((D, 2 * TCC), lambda g: (0, g)),
            pl.BlockSpec((D, 2 * TCC), lambda g: (0, g)),
        ],
        out_specs=[
            pl.BlockSpec((TCC, 2 * D), lambda g: (g, 0)),
            pl.BlockSpec((TCC, 2 * D), lambda g: (g, 0)),
        ],
        out_shape=[
            jax.ShapeDtypeStruct((H, 2 * D), jnp.float32),
            jax.ShapeDtypeStruct((H, 2 * D), jnp.float32),
        ],
    )(ut_t, it_t)


def kernel(user_table, item_table, user_idx, item_idx):
    B = user_idx.shape[0]
    N, D = user_table.shape
    bpw = B // NW
    nchunk = bpw // CHUNK
    nphase = bpw // PHROWS
    cpp = PHROWS // CHUNK

    ut2, it2 = _repack(user_table.T, item_table.T)

    ui32 = user_idx.astype(jnp.int32)
    ii32 = item_idx.astype(jnp.int32)
    def _decompose(r):
        g, loc = r // (2 * TCC), r % (2 * TCC)
        return g * TCC + loc % TCC, loc // TCC

    ubig, uh = _decompose(ui32)
    ibig, ih = _decompose(ii32)
    ub2 = ubig.reshape(NW * nchunk, CHUNK)
    ib2 = ibig.reshape(NW * nchunk, CHUNK)

    mesh = plsc.VectorSubcoreMesh(core_axis_name="c", subcore_axis_name="s")

    @functools.partial(
        pl.kernel,
        mesh=mesh,
        compiler_params=pltpu.CompilerParams(needs_layout_passes=False),
        out_type=jax.ShapeDtypeStruct((B,), jnp.float32),
        scratch_types=[
            pltpu.VMEM((nchunk, CHUNK), jnp.int32),
            pltpu.VMEM((nchunk, CHUNK), jnp.int32),
            pltpu.VMEM((bpw,), jnp.int32),
            pltpu.VMEM((bpw,), jnp.int32),
            pltpu.VMEM((PHROWS, 2 * D), jnp.float32),
            pltpu.VMEM((PHROWS, 2 * D), jnp.float32),
            pltpu.VMEM((bpw,), jnp.float32),
            pltpu.SemaphoreType.DMA,
            pltpu.SemaphoreType.DMA,
        ],
    )
    def _k(ut_hbm, it_hbm, ub_hbm, ib_hbm, uh_hbm, ih_hbm, out_hbm,
           ub_v, ib_v, uh_v, ih_v, ur_v, ir_v, o_v, sem_u, sem_i):
        wid = lax.axis_index("s") * NC + lax.axis_index("c")
        pltpu.sync_copy(ub_hbm.at[pl.ds(wid * nchunk, nchunk)], ub_v)
        pltpu.sync_copy(ib_hbm.at[pl.ds(wid * nchunk, nchunk)], ib_v)
        pltpu.sync_copy(uh_hbm.at[pl.ds(wid * bpw, bpw)], uh_v)
        pltpu.sync_copy(ih_hbm.at[pl.ds(wid * bpw, bpw)], ih_v)

        for h in range(nphase):
            copies = []
            for j in range(cpp):
                copies.append(pltpu.async_copy(
                    ut_hbm.at[ub_v.at[h * cpp + j]],
                    ur_v.at[pl.ds(j * CHUNK, CHUNK)], sem_u))
                copies.append(pltpu.async_copy(
                    it_hbm.at[ib_v.at[h * cpp + j]],
                    ir_v.at[pl.ds(j * CHUNK, CHUNK)], sem_i))
            for c in copies:
                c.wait()

            def group(g, carry):
                rows = g * L + lax.broadcasted_iota(jnp.int32, (L,), 0)
                boff = h * PHROWS + g * L
                ucol = uh_v[pl.ds(boff, L)] * D
                icol = ih_v[pl.ds(boff, L)] * D
                acc = jnp.zeros((L,), jnp.float32)
                for d in range(D):
                    u = plsc.load_gather(ur_v, [rows, ucol + d])
                    v = plsc.load_gather(ir_v, [rows, icol + d])
                    acc = acc + u * v
                o_v[pl.ds(boff, L)] = acc
                return carry

            lax.fori_loop(0, PHROWS // L, group, 0)

        pltpu.sync_copy(o_v, out_hbm.at[pl.ds(wid * bpw, bpw)])

    return _k(ut2, it2, ub2, ib2, uh, ih)


# speedup vs baseline: 2.9567x; 1.2398x over previous
import functools

import jax
import jax.numpy as jnp
from jax import lax
from jax.experimental import pallas as pl
from jax.experimental.pallas import tpu as pltpu
from jax.experimental.pallas import tpu_sc as plsc

NC = 2
NS = 16
L = 16
NW = NC * NS
CHUNK = 128
PHROWS = 256
TCC = 4096


def _repack_kernel(u, i, uo, io):
    uo[...] = jnp.concatenate([u[:, :TCC], u[:, TCC:]], axis=0).T
    io[...] = jnp.concatenate([i[:, :TCC], i[:, TCC:]], axis=0).T


def _repack(ut_t, it_t):
    D, N = ut_t.shape
    grid = (N // 2 + TCC - 1) // TCC
    H = grid * TCC
    return pl.pallas_call(
        _repack_kernel,
        grid=(grid,),
        in_specs=[
            pl.BlockSpec((D, 2 * TCC), lambda g: (0, g)),
            pl.BlockSpec((D, 2 * TCC), lambda g: (0, g)),
        ],
        out_specs=[
            pl.BlockSpec((TCC, 2 * D), lambda g: (g, 0)),
            pl.BlockSpec((TCC, 2 * D), lambda g: (g, 0)),
        ],
        out_shape=[
            jax.ShapeDtypeStruct((H, 2 * D), jnp.float32),
            jax.ShapeDtypeStruct((H, 2 * D), jnp.float32),
        ],
    )(ut_t, it_t)


def kernel(user_table, item_table, user_idx, item_idx):
    B = user_idx.shape[0]
    N, D = user_table.shape
    bpw = B // NW
    nchunk = bpw // CHUNK
    nphase = bpw // PHROWS
    cpp = PHROWS // CHUNK

    ut2, it2 = _repack(user_table.T, item_table.T)

    ui32 = user_idx.astype(jnp.int32)
    ii32 = item_idx.astype(jnp.int32)
    def _decompose(r):
        g, loc = r // (2 * TCC), r % (2 * TCC)
        return g * TCC + loc % TCC, loc // TCC

    ubig, uh = _decompose(ui32)
    ibig, ih = _decompose(ii32)
    ub2 = ubig.reshape(NW * nchunk, CHUNK)
    ib2 = ibig.reshape(NW * nchunk, CHUNK)

    mesh = plsc.VectorSubcoreMesh(core_axis_name="c", subcore_axis_name="s")

    @functools.partial(
        pl.kernel,
        mesh=mesh,
        compiler_params=pltpu.CompilerParams(needs_layout_passes=False),
        out_type=jax.ShapeDtypeStruct((B,), jnp.float32),
        scratch_types=[
            pltpu.VMEM((nchunk, CHUNK), jnp.int32),
            pltpu.VMEM((nchunk, CHUNK), jnp.int32),
            pltpu.VMEM((bpw,), jnp.int32),
            pltpu.VMEM((bpw,), jnp.int32),
            pltpu.VMEM((PHROWS, 2 * D), jnp.float32),
            pltpu.VMEM((PHROWS, 2 * D), jnp.float32),
            pltpu.VMEM((bpw,), jnp.float32),
            pltpu.SemaphoreType.DMA,
            pltpu.SemaphoreType.DMA,
        ],
    )
    def _k(ut_hbm, it_hbm, ub_hbm, ib_hbm, uh_hbm, ih_hbm, out_hbm,
           ub_v, ib_v, uh_v, ih_v, ur_v, ir_v, o_v, sem_u, sem_i):
        wid = lax.axis_index("s") * NC + lax.axis_index("c")
        pltpu.sync_copy(ub_hbm.at[pl.ds(wid * nchunk, nchunk)], ub_v)
        pltpu.sync_copy(ib_hbm.at[pl.ds(wid * nchunk, nchunk)], ib_v)
        pltpu.sync_copy(uh_hbm.at[pl.ds(wid * bpw, bpw)], uh_v)
        pltpu.sync_copy(ih_hbm.at[pl.ds(wid * bpw, bpw)], ih_v)

        for h in range(nphase):
            copies = []
            for j in range(cpp):
                copies.append(pltpu.async_copy(
                    ut_hbm.at[ub_v.at[h * cpp + j]],
                    ur_v.at[pl.ds(j * CHUNK, CHUNK)], sem_u))
                copies.append(pltpu.async_copy(
                    it_hbm.at[ib_v.at[h * cpp + j]],
                    ir_v.at[pl.ds(j * CHUNK, CHUNK)], sem_i))
            for c in copies:
                c.wait()

            def group(g, carry):
                rows = g * L + lax.broadcasted_iota(jnp.int32, (L,), 0)
                boff = h * PHROWS + g * L
                ucol = uh_v[pl.ds(boff, L)] * D
                icol = ih_v[pl.ds(boff, L)] * D
                acc = jnp.zeros((L,), jnp.float32)
                for d in range(D):
                    u = plsc.load_gather(ur_v, [rows, ucol + d])
                    v = plsc.load_gather(ir_v, [rows, icol + d])
                    acc = acc + u * v
                o_v[pl.ds(boff, L)] = acc
                return carry

            lax.fori_loop(0, PHROWS // L, group, 0)

        pltpu.sync_copy(o_v, out_hbm.at[pl.ds(wid * bpw, bpw)])

    return _k(ut2, it2, ub2, ib2, uh, ih)


# speedup vs baseline: 2.9781x; 1.0072x over previous
import functools

import jax
import jax.numpy as jnp
from jax import lax
from jax.experimental import pallas as pl
from jax.experimental.pallas import tpu as pltpu
from jax.experimental.pallas import tpu_sc as plsc

NC = 2
NS = 16
L = 16
NW = NC * NS
CHUNK = 128
TCC = 4096


def _repack_kernel(u, i, uo, io):
    uo[...] = jnp.concatenate([u[:, :TCC], u[:, TCC:]], axis=0).T
    io[...] = jnp.concatenate([i[:, :TCC], i[:, TCC:]], axis=0).T


def _repack(ut_t, it_t):
    D, N = ut_t.shape
    grid = (N // 2 + TCC - 1) // TCC
    H = grid * TCC
    return pl.pallas_call(
        _repack_kernel,
        grid=(grid,),
        in_specs=[
            pl.BlockSpec((D, 2 * TCC), lambda g: (0, g)),
            pl.BlockSpec((D, 2 * TCC), lambda g: (0, g)),
        ],
        out_specs=[
            pl.BlockSpec((TCC, 2 * D), lambda g: (g, 0)),
            pl.BlockSpec((TCC, 2 * D), lambda g: (g, 0)),
        ],
        out_shape=[
            jax.ShapeDtypeStruct((H, 2 * D), jnp.float32),
            jax.ShapeDtypeStruct((H, 2 * D), jnp.float32),
        ],
    )(ut_t, it_t)


def kernel(user_table, item_table, user_idx, item_idx):
    B = user_idx.shape[0]
    N, D = user_table.shape
    bpw = B // NW
    nchunk = bpw // CHUNK

    ut2, it2 = _repack(user_table.T, item_table.T)

    ui32 = user_idx.astype(jnp.int32)
    ii32 = item_idx.astype(jnp.int32)
    def _decompose(r):
        g, loc = r // (2 * TCC), r % (2 * TCC)
        return g * TCC + loc % TCC, loc // TCC

    ubig, uh = _decompose(ui32)
    ibig, ih = _decompose(ii32)
    ub2 = ubig.reshape(NW * nchunk, CHUNK)
    ib2 = ibig.reshape(NW * nchunk, CHUNK)

    mesh = plsc.VectorSubcoreMesh(core_axis_name="c", subcore_axis_name="s")

    @functools.partial(
        pl.kernel,
        mesh=mesh,
        compiler_params=pltpu.CompilerParams(needs_layout_passes=False),
        out_type=jax.ShapeDtypeStruct((B,), jnp.float32),
        scratch_types=[
            pltpu.VMEM((nchunk, CHUNK), jnp.int32),
            pltpu.VMEM((nchunk, CHUNK), jnp.int32),
            pltpu.VMEM((bpw,), jnp.int32),
            pltpu.VMEM((bpw,), jnp.int32),
            pltpu.VMEM((2, CHUNK, 2 * D), jnp.float32),
            pltpu.VMEM((2, CHUNK, 2 * D), jnp.float32),
            pltpu.VMEM((bpw,), jnp.float32),
            pltpu.SemaphoreType.DMA,
            pltpu.SemaphoreType.DMA,
            pltpu.SemaphoreType.DMA,
            pltpu.SemaphoreType.DMA,
        ],
    )
    def _k(ut_hbm, it_hbm, ub_hbm, ib_hbm, uh_hbm, ih_hbm, out_hbm,
           ub_v, ib_v, uh_v, ih_v, ur_v, ir_v, o_v,
           sem_u0, sem_u1, sem_i0, sem_i1):
        wid = lax.axis_index("s") * NC + lax.axis_index("c")
        pltpu.sync_copy(ub_hbm.at[pl.ds(wid * nchunk, nchunk)], ub_v)
        pltpu.sync_copy(ib_hbm.at[pl.ds(wid * nchunk, nchunk)], ib_v)
        pltpu.sync_copy(uh_hbm.at[pl.ds(wid * bpw, bpw)], uh_v)
        pltpu.sync_copy(ih_hbm.at[pl.ds(wid * bpw, bpw)], ih_v)

        sems_u = (sem_u0, sem_u1)
        sems_i = (sem_i0, sem_i1)

        def issue(j):
            slot = j % 2
            cu = pltpu.async_copy(
                ut_hbm.at[ub_v.at[j]], ur_v.at[slot], sems_u[slot])
            ci = pltpu.async_copy(
                it_hbm.at[ib_v.at[j]], ir_v.at[slot], sems_i[slot])
            return cu, ci

        pending = issue(0)
        for j in range(nchunk):
            nxt = issue(j + 1) if j + 1 < nchunk else None
            for c in pending:
                c.wait()
            slot = j % 2

            def group(g, carry):
                rows = g * L + lax.broadcasted_iota(jnp.int32, (L,), 0)
                ucol = uh_v[pl.ds(j * CHUNK + g * L, L)] * D
                icol = ih_v[pl.ds(j * CHUNK + g * L, L)] * D
                acc = jnp.zeros((L,), jnp.float32)
                for d in range(D):
                    u = plsc.load_gather(ur_v.at[slot], [rows, ucol + d])
                    v = plsc.load_gather(ir_v.at[slot], [rows, icol + d])
                    acc = acc + u * v
                o_v[pl.ds(j * CHUNK + g * L, L)] = acc
                return carry

            lax.fori_loop(0, CHUNK // L, group, 0)
            pending = nxt

        pltpu.sync_copy(o_v, out_hbm.at[pl.ds(wid * bpw, bpw)])

    return _k(ut2, it2, ub2, ib2, uh, ih)


# speedup vs baseline: 3.0307x; 1.0176x over previous
import functools

import jax
import jax.numpy as jnp
from jax import lax
from jax.experimental import pallas as pl
from jax.experimental.pallas import tpu as pltpu
from jax.experimental.pallas import tpu_sc as plsc

NC = 2
NS = 16
L = 16
NW = NC * NS
CHUNK = 128
TCC = 8192


def _repack_kernel(u, i, uo, io):
    uo[...] = jnp.concatenate([u[:, :TCC], u[:, TCC:]], axis=0).T
    io[...] = jnp.concatenate([i[:, :TCC], i[:, TCC:]], axis=0).T


def _repack(ut_t, it_t):
    D, N = ut_t.shape
    grid = (N // 2 + TCC - 1) // TCC
    H = grid * TCC
    return pl.pallas_call(
        _repack_kernel,
        grid=(grid,),
        in_specs=[
            pl.BlockSpec((D, 2 * TCC), lambda g: (0, g)),
            pl.BlockSpec((D, 2 * TCC), lambda g: (0, g)),
        ],
        out_specs=[
            pl.BlockSpec((TCC, 2 * D), lambda g: (g, 0)),
            pl.BlockSpec((TCC, 2 * D), lambda g: (g, 0)),
        ],
        out_shape=[
            jax.ShapeDtypeStruct((H, 2 * D), jnp.float32),
            jax.ShapeDtypeStruct((H, 2 * D), jnp.float32),
        ],
    )(ut_t, it_t)


def kernel(user_table, item_table, user_idx, item_idx):
    B = user_idx.shape[0]
    N, D = user_table.shape
    bpw = B // NW
    nchunk = bpw // CHUNK

    ut2, it2 = _repack(user_table.T, item_table.T)

    ui32 = user_idx.astype(jnp.int32)
    ii32 = item_idx.astype(jnp.int32)
    def _decompose(r):
        g, loc = r // (2 * TCC), r % (2 * TCC)
        return g * TCC + loc % TCC, loc // TCC

    ubig, uh = _decompose(ui32)
    ibig, ih = _decompose(ii32)
    ub2 = ubig.reshape(NW * nchunk, CHUNK)
    ib2 = ibig.reshape(NW * nchunk, CHUNK)

    mesh = plsc.VectorSubcoreMesh(core_axis_name="c", subcore_axis_name="s")

    @functools.partial(
        pl.kernel,
        mesh=mesh,
        compiler_params=pltpu.CompilerParams(needs_layout_passes=False),
        out_type=jax.ShapeDtypeStruct((B,), jnp.float32),
        scratch_types=[
            pltpu.VMEM((nchunk, CHUNK), jnp.int32),
            pltpu.VMEM((nchunk, CHUNK), jnp.int32),
            pltpu.VMEM((bpw,), jnp.int32),
            pltpu.VMEM((bpw,), jnp.int32),
            pltpu.VMEM((2, CHUNK, 2 * D), jnp.float32),
            pltpu.VMEM((2, CHUNK, 2 * D), jnp.float32),
            pltpu.VMEM((bpw,), jnp.float32),
            pltpu.SemaphoreType.DMA,
            pltpu.SemaphoreType.DMA,
            pltpu.SemaphoreType.DMA,
            pltpu.SemaphoreType.DMA,
        ],
    )
    def _k(ut_hbm, it_hbm, ub_hbm, ib_hbm, uh_hbm, ih_hbm, out_hbm,
           ub_v, ib_v, uh_v, ih_v, ur_v, ir_v, o_v,
           sem_u0, sem_u1, sem_i0, sem_i1):
        wid = lax.axis_index("s") * NC + lax.axis_index("c")
        pltpu.sync_copy(ub_hbm.at[pl.ds(wid * nchunk, nchunk)], ub_v)
        pltpu.sync_copy(ib_hbm.at[pl.ds(wid * nchunk, nchunk)], ib_v)
        pltpu.sync_copy(uh_hbm.at[pl.ds(wid * bpw, bpw)], uh_v)
        pltpu.sync_copy(ih_hbm.at[pl.ds(wid * bpw, bpw)], ih_v)

        sems_u = (sem_u0, sem_u1)
        sems_i = (sem_i0, sem_i1)

        def issue(j):
            slot = j % 2
            cu = pltpu.async_copy(
                ut_hbm.at[ub_v.at[j]], ur_v.at[slot], sems_u[slot])
            ci = pltpu.async_copy(
                it_hbm.at[ib_v.at[j]], ir_v.at[slot], sems_i[slot])
            return cu, ci

        pending = issue(0)
        for j in range(nchunk):
            nxt = issue(j + 1) if j + 1 < nchunk else None
            for c in pending:
                c.wait()
            slot = j % 2

            def group(g, carry):
                rows = g * L + lax.broadcasted_iota(jnp.int32, (L,), 0)
                ucol = uh_v[pl.ds(j * CHUNK + g * L, L)] * D
                icol = ih_v[pl.ds(j * CHUNK + g * L, L)] * D
                acc = jnp.zeros((L,), jnp.float32)
                for d in range(D):
                    u = plsc.load_gather(ur_v.at[slot], [rows, ucol + d])
                    v = plsc.load_gather(ir_v.at[slot], [rows, icol + d])
                    acc = acc + u * v
                o_v[pl.ds(j * CHUNK + g * L, L)] = acc
                return carry

            lax.fori_loop(0, CHUNK // L, group, 0)
            pending = nxt

        pltpu.sync_copy(o_v, out_hbm.at[pl.ds(wid * bpw, bpw)])

    return _k(ut2, it2, ub2, ib2, uh, ih)


# speedup vs baseline: 3.0496x; 1.0063x over previous
import functools

import jax
import jax.numpy as jnp
from jax import lax
from jax.experimental import pallas as pl
from jax.experimental.pallas import tpu as pltpu
from jax.experimental.pallas import tpu_sc as plsc

NC = 2
NS = 16
L = 16
NW = NC * NS
CHUNK = 128
TCC = 12288


def _repack_kernel(u, i, uo, io):
    uo[...] = jnp.concatenate([u[:, :TCC], u[:, TCC:]], axis=0).T
    io[...] = jnp.concatenate([i[:, :TCC], i[:, TCC:]], axis=0).T


def _repack(ut_t, it_t):
    D, N = ut_t.shape
    grid = (N // 2 + TCC - 1) // TCC
    H = grid * TCC
    return pl.pallas_call(
        _repack_kernel,
        grid=(grid,),
        in_specs=[
            pl.BlockSpec((D, 2 * TCC), lambda g: (0, g)),
            pl.BlockSpec((D, 2 * TCC), lambda g: (0, g)),
        ],
        out_specs=[
            pl.BlockSpec((TCC, 2 * D), lambda g: (g, 0)),
            pl.BlockSpec((TCC, 2 * D), lambda g: (g, 0)),
        ],
        out_shape=[
            jax.ShapeDtypeStruct((H, 2 * D), jnp.float32),
            jax.ShapeDtypeStruct((H, 2 * D), jnp.float32),
        ],
    )(ut_t, it_t)


def kernel(user_table, item_table, user_idx, item_idx):
    B = user_idx.shape[0]
    N, D = user_table.shape
    bpw = B // NW
    nchunk = bpw // CHUNK

    ut2, it2 = _repack(user_table.T, item_table.T)

    ui32 = user_idx.astype(jnp.int32)
    ii32 = item_idx.astype(jnp.int32)
    def _decompose(r):
        g, loc = r // (2 * TCC), r % (2 * TCC)
        return g * TCC + loc % TCC, loc // TCC

    ubig, uh = _decompose(ui32)
    ibig, ih = _decompose(ii32)
    ub2 = ubig.reshape(NW * nchunk, CHUNK)
    ib2 = ibig.reshape(NW * nchunk, CHUNK)

    mesh = plsc.VectorSubcoreMesh(core_axis_name="c", subcore_axis_name="s")

    @functools.partial(
        pl.kernel,
        mesh=mesh,
        compiler_params=pltpu.CompilerParams(needs_layout_passes=False),
        out_type=jax.ShapeDtypeStruct((B,), jnp.float32),
        scratch_types=[
            pltpu.VMEM((nchunk, CHUNK), jnp.int32),
            pltpu.VMEM((nchunk, CHUNK), jnp.int32),
            pltpu.VMEM((bpw,), jnp.int32),
            pltpu.VMEM((bpw,), jnp.int32),
            pltpu.VMEM((2, CHUNK, 2 * D), jnp.float32),
            pltpu.VMEM((2, CHUNK, 2 * D), jnp.float32),
            pltpu.VMEM((bpw,), jnp.float32),
            pltpu.SemaphoreType.DMA,
            pltpu.SemaphoreType.DMA,
            pltpu.SemaphoreType.DMA,
            pltpu.SemaphoreType.DMA,
        ],
    )
    def _k(ut_hbm, it_hbm, ub_hbm, ib_hbm, uh_hbm, ih_hbm, out_hbm,
           ub_v, ib_v, uh_v, ih_v, ur_v, ir_v, o_v,
           sem_u0, sem_u1, sem_i0, sem_i1):
        wid = lax.axis_index("s") * NC + lax.axis_index("c")
        pltpu.sync_copy(ub_hbm.at[pl.ds(wid * nchunk, nchunk)], ub_v)
        pltpu.sync_copy(ib_hbm.at[pl.ds(wid * nchunk, nchunk)], ib_v)
        pltpu.sync_copy(uh_hbm.at[pl.ds(wid * bpw, bpw)], uh_v)
        pltpu.sync_copy(ih_hbm.at[pl.ds(wid * bpw, bpw)], ih_v)

        sems_u = (sem_u0, sem_u1)
        sems_i = (sem_i0, sem_i1)

        def issue(j):
            slot = j % 2
            cu = pltpu.async_copy(
                ut_hbm.at[ub_v.at[j]], ur_v.at[slot], sems_u[slot])
            ci = pltpu.async_copy(
                it_hbm.at[ib_v.at[j]], ir_v.at[slot], sems_i[slot])
            return cu, ci

        pending = issue(0)
        for j in range(nchunk):
            nxt = issue(j + 1) if j + 1 < nchunk else None
            for c in pending:
                c.wait()
            slot = j % 2

            def group(g, carry):
                rows = g * L + lax.broadcasted_iota(jnp.int32, (L,), 0)
                ucol = uh_v[pl.ds(j * CHUNK + g * L, L)] * D
                icol = ih_v[pl.ds(j * CHUNK + g * L, L)] * D
                acc = jnp.zeros((L,), jnp.float32)
                for d in range(D):
                    u = plsc.load_gather(ur_v.at[slot], [rows, ucol + d])
                    v = plsc.load_gather(ir_v.at[slot], [rows, icol + d])
                    acc = acc + u * v
                o_v[pl.ds(j * CHUNK + g * L, L)] = acc
                return carry

            lax.fori_loop(0, CHUNK // L, group, 0)
            pending = nxt

        pltpu.sync_copy(o_v, out_hbm.at[pl.ds(wid * bpw, bpw)])

    return _k(ut2, it2, ub2, ib2, uh, ih)


# speedup vs baseline: 3.0505x; 1.0003x over previous
import functools

import jax
import jax.numpy as jnp
from jax import lax
from jax.experimental import pallas as pl
from jax.experimental.pallas import tpu as pltpu
from jax.experimental.pallas import tpu_sc as plsc

NC = 2
NS = 16
L = 16
NW = NC * NS
CHUNK = 128
TCC = 14336


def _repack_kernel(u, i, uo, io):
    uo[...] = jnp.concatenate([u[:, :TCC], u[:, TCC:]], axis=0).T
    io[...] = jnp.concatenate([i[:, :TCC], i[:, TCC:]], axis=0).T


def _repack(ut_t, it_t):
    D, N = ut_t.shape
    grid = (N // 2 + TCC - 1) // TCC
    H = grid * TCC
    return pl.pallas_call(
        _repack_kernel,
        grid=(grid,),
        in_specs=[
            pl.BlockSpec((D, 2 * TCC), lambda g: (0, g)),
            pl.BlockSpec((D, 2 * TCC), lambda g: (0, g)),
        ],
        out_specs=[
            pl.BlockSpec((TCC, 2 * D), lambda g: (g, 0)),
            pl.BlockSpec((TCC, 2 * D), lambda g: (g, 0)),
        ],
        out_shape=[
            jax.ShapeDtypeStruct((H, 2 * D), jnp.float32),
            jax.ShapeDtypeStruct((H, 2 * D), jnp.float32),
        ],
    )(ut_t, it_t)


def kernel(user_table, item_table, user_idx, item_idx):
    B = user_idx.shape[0]
    N, D = user_table.shape
    bpw = B // NW
    nchunk = bpw // CHUNK

    ut2, it2 = _repack(user_table.T, item_table.T)

    ui32 = user_idx.astype(jnp.int32)
    ii32 = item_idx.astype(jnp.int32)
    def _decompose(r):
        g, loc = r // (2 * TCC), r % (2 * TCC)
        return g * TCC + loc % TCC, loc // TCC

    ubig, uh = _decompose(ui32)
    ibig, ih = _decompose(ii32)
    ub2 = ubig.reshape(NW * nchunk, CHUNK)
    ib2 = ibig.reshape(NW * nchunk, CHUNK)

    mesh = plsc.VectorSubcoreMesh(core_axis_name="c", subcore_axis_name="s")

    @functools.partial(
        pl.kernel,
        mesh=mesh,
        compiler_params=pltpu.CompilerParams(needs_layout_passes=False),
        out_type=jax.ShapeDtypeStruct((B,), jnp.float32),
        scratch_types=[
            pltpu.VMEM((nchunk, CHUNK), jnp.int32),
            pltpu.VMEM((nchunk, CHUNK), jnp.int32),
            pltpu.VMEM((bpw,), jnp.int32),
            pltpu.VMEM((bpw,), jnp.int32),
            pltpu.VMEM((2, CHUNK, 2 * D), jnp.float32),
            pltpu.VMEM((2, CHUNK, 2 * D), jnp.float32),
            pltpu.VMEM((bpw,), jnp.float32),
            pltpu.SemaphoreType.DMA,
            pltpu.SemaphoreType.DMA,
            pltpu.SemaphoreType.DMA,
            pltpu.SemaphoreType.DMA,
        ],
    )
    def _k(ut_hbm, it_hbm, ub_hbm, ib_hbm, uh_hbm, ih_hbm, out_hbm,
           ub_v, ib_v, uh_v, ih_v, ur_v, ir_v, o_v,
           sem_u0, sem_u1, sem_i0, sem_i1):
        wid = lax.axis_index("s") * NC + lax.axis_index("c")
        pltpu.sync_copy(ub_hbm.at[pl.ds(wid * nchunk, nchunk)], ub_v)
        pltpu.sync_copy(ib_hbm.at[pl.ds(wid * nchunk, nchunk)], ib_v)
        pltpu.sync_copy(uh_hbm.at[pl.ds(wid * bpw, bpw)], uh_v)
        pltpu.sync_copy(ih_hbm.at[pl.ds(wid * bpw, bpw)], ih_v)

        sems_u = (sem_u0, sem_u1)
        sems_i = (sem_i0, sem_i1)

        def issue(j):
            slot = j % 2
            cu = pltpu.async_copy(
                ut_hbm.at[ub_v.at[j]], ur_v.at[slot], sems_u[slot])
            ci = pltpu.async_copy(
                it_hbm.at[ib_v.at[j]], ir_v.at[slot], sems_i[slot])
            return cu, ci

        pending = issue(0)
        for j in range(nchunk):
            nxt = issue(j + 1) if j + 1 < nchunk else None
            for c in pending:
                c.wait()
            slot = j % 2

            def group(g, carry):
                rows = g * L + lax.broadcasted_iota(jnp.int32, (L,), 0)
                ucol = uh_v[pl.ds(j * CHUNK + g * L, L)] * D
                icol = ih_v[pl.ds(j * CHUNK + g * L, L)] * D
                acc = jnp.zeros((L,), jnp.float32)
                for d in range(D):
                    u = plsc.load_gather(ur_v.at[slot], [rows, ucol + d])
                    v = plsc.load_gather(ir_v.at[slot], [rows, icol + d])
                    acc = acc + u * v
                o_v[pl.ds(j * CHUNK + g * L, L)] = acc
                return carry

            lax.fori_loop(0, CHUNK // L, group, 0)
            pending = nxt

        pltpu.sync_copy(o_v, out_hbm.at[pl.ds(wid * bpw, bpw)])

    return _k(ut2, it2, ub2, ib2, uh, ih)
